# baseline (speedup 1.0000x reference)
import functools

import jax
import jax.numpy as jnp
from jax import lax
from jax.experimental import pallas as pl
from jax.experimental.pallas import tpu as pltpu
from jax.experimental.pallas import tpu_sc as plsc


S_BLK = 2048

_NC = 2
_NS = 16
_NW = _NC * _NS
_SC_ROWS = 8192
_RPW = _SC_ROWS // _NW
_CH = 32
_NCH = _RPW // _CH
_D = 1024


def _sc_body(x_hbm, pos_hbm, out_hbm, xbuf, pbuf, sem1, sem2):
    wid = lax.axis_index("s") * _NC + lax.axis_index("c")
    base = wid * _RPW

    def chunk(c, carry):
        r0 = base + c * _CH
        cp1 = pltpu.async_copy(x_hbm.at[pl.ds(3 * 8192 + r0, _CH)], xbuf, sem1)
        cp2 = pltpu.async_copy(pos_hbm.at[pl.ds(r0, _CH)], pbuf, sem2)
        cp1.wait()
        cp2.wait()

        def add_row(r, carry2):
            for j in range(_D // 16):
                sl = pl.ds(j * 16, 16)
                xbuf[r, sl] = xbuf[r, sl] + pbuf[r, sl]
            return carry2

        lax.fori_loop(0, _CH, add_row, 0)
        pltpu.sync_copy(xbuf, out_hbm.at[pl.ds(r0, _CH)])
        return carry

    lax.fori_loop(0, _NCH, chunk, 0)


_sc_add = functools.partial(
    pl.kernel,
    mesh=plsc.VectorSubcoreMesh(core_axis_name="c", subcore_axis_name="s"),
    out_type=jax.ShapeDtypeStruct((_SC_ROWS, _D), jnp.float32),
    scratch_types=[
        pltpu.VMEM((_CH, _D), jnp.float32),
        pltpu.VMEM((_CH, _D), jnp.float32),
        pltpu.SemaphoreType.DMA,
        pltpu.SemaphoreType.DMA,
    ],
)(_sc_body)


def _add_kernel(x_ref, pos_ref, o_ref):
    o_ref[...] = x_ref[...] + pos_ref[...]


def kernel(x, pos_table):
    B, S, D = x.shape
    pos = pos_table[:S]
    x_flat = x.reshape(B * S, D)

    sc_out = _sc_add(x_flat, pos)

    tc_out = pl.pallas_call(
        _add_kernel,
        grid=(S // S_BLK, B - 1),
        in_specs=[
            pl.BlockSpec((1, S_BLK, D), lambda s, b: (b, s, 0)),
            pl.BlockSpec((S_BLK, D), lambda s, b: (s, 0)),
        ],
        out_specs=pl.BlockSpec((1, S_BLK, D), lambda s, b: (b, s, 0)),
        out_shape=jax.ShapeDtypeStruct((B - 1, S, D), x.dtype),
        compiler_params=pltpu.CompilerParams(
            dimension_semantics=("parallel", "parallel"),
        ),
    )(x, pos)

    return jnp.concatenate([tc_out, sc_out.reshape(1, S, D)], axis=0)


# speedup vs baseline: 2.2147x; 2.2147x over previous
import jax
import jax.numpy as jnp
from jax.experimental import pallas as pl
from jax.experimental.pallas import tpu as pltpu


S_BLK = 2048


def _add_kernel(x_ref, pos_ref, o_ref):
    o_ref[...] = x_ref[...] + pos_ref[...]


def kernel(x, pos_table):
    B, S, D = x.shape
    grid = (S // S_BLK, B)
    return pl.pallas_call(
        _add_kernel,
        grid=grid,
        in_specs=[
            pl.BlockSpec((1, S_BLK, D), lambda s, b: (b, s, 0)),
            pl.BlockSpec((S_BLK, D), lambda s, b: (s, 0)),
        ],
        out_specs=pl.BlockSpec((1, S_BLK, D), lambda s, b: (b, s, 0)),
        out_shape=jax.ShapeDtypeStruct((B, S, D), x.dtype),
        compiler_params=pltpu.CompilerParams(
            dimension_semantics=("parallel", "parallel"),
        ),
    )(x, pos_table[:S])
